# baseline (speedup 1.0000x reference)
import jax
import jax.numpy as jnp
from jax.experimental import pallas as pl
from jax.experimental.pallas import tpu as pltpu

KB = 128


def _argmax_kernel(x_ref, am_ref):
    xb = x_ref[...]
    K = xb.shape[1]
    mx = jnp.max(xb, axis=1, keepdims=True)
    iota = jax.lax.broadcasted_iota(jnp.int32, xb.shape, 1)
    cand = jnp.where(xb == mx, iota, K)
    am = jnp.min(cand, axis=1, keepdims=True)
    am_ref[...] = jnp.broadcast_to(am, am_ref.shape)


def _onehot_kernel(am_ref, o_ref):
    k0 = pl.program_id(1) * KB
    am = am_ref[:, 0:1, :]
    iota = k0 + jax.lax.broadcasted_iota(jnp.int32, o_ref.shape, 1)
    o_ref[...] = (iota == am).astype(jnp.float32)


def kernel(x):
    B, Kdim, L = x.shape
    am = pl.pallas_call(
        _argmax_kernel,
        grid=(B,),
        in_specs=[pl.BlockSpec((1, Kdim, L), lambda b: (b, 0, 0))],
        out_specs=pl.BlockSpec((1, 8, L), lambda b: (b, 0, 0)),
        out_shape=jax.ShapeDtypeStruct((B, 8, L), jnp.int32),
        compiler_params=pltpu.CompilerParams(
            dimension_semantics=("parallel",),
        ),
    )(x)
    out = pl.pallas_call(
        _onehot_kernel,
        grid=(B, Kdim // KB),
        in_specs=[pl.BlockSpec((1, 8, L), lambda b, k: (b, 0, 0))],
        out_specs=pl.BlockSpec((1, KB, L), lambda b, k: (b, k, 0)),
        out_shape=jax.ShapeDtypeStruct((B, Kdim, L), x.dtype),
        compiler_params=pltpu.CompilerParams(
            dimension_semantics=("parallel", "parallel"),
        ),
    )(am)
    return out


# speedup vs baseline: 1.4105x; 1.4105x over previous
import jax
import jax.numpy as jnp
from jax.experimental import pallas as pl
from jax.experimental.pallas import tpu as pltpu

BB = 1


def _stargmax_kernel(x_ref, o_ref):
    xb = x_ref[...]
    K = xb.shape[1]
    mx = jnp.max(xb, axis=1, keepdims=True)
    iota = jax.lax.broadcasted_iota(jnp.int32, xb.shape, 1)
    cand = jnp.where(xb == mx, iota, K)
    am = jnp.min(cand, axis=1, keepdims=True)
    o_ref[...] = (iota == am).astype(jnp.float32)


def kernel(x):
    B, Kdim, L = x.shape
    grid = (B // BB,)
    return pl.pallas_call(
        _stargmax_kernel,
        grid=grid,
        in_specs=[pl.BlockSpec((BB, Kdim, L), lambda b: (b, 0, 0))],
        out_specs=pl.BlockSpec((BB, Kdim, L), lambda b: (b, 0, 0)),
        out_shape=jax.ShapeDtypeStruct((B, Kdim, L), x.dtype),
        compiler_params=pltpu.CompilerParams(
            dimension_semantics=("parallel",),
        ),
    )(x)


# speedup vs baseline: 1.4608x; 1.0357x over previous
import jax
import jax.numpy as jnp
from jax.experimental import pallas as pl
from jax.experimental.pallas import tpu as pltpu

BB = 4


def _stargmax_kernel(x_ref, o_ref):
    xb = x_ref[...]
    K = xb.shape[1]
    mx = jnp.max(xb, axis=1, keepdims=True)
    iota_i = jax.lax.broadcasted_iota(jnp.int32, xb.shape, 1) + 0x4B000000
    iota_f = jax.lax.bitcast_convert_type(iota_i, jnp.float32)
    big = jnp.float32(float(2 ** 23 + K))
    cand = jnp.where(xb == mx, iota_f, big)
    am = jnp.min(cand, axis=1, keepdims=True)
    o_ref[...] = (iota_f == am).astype(jnp.float32)


def kernel(x):
    B, Kdim, L = x.shape
    grid = (B // BB,)
    return pl.pallas_call(
        _stargmax_kernel,
        grid=grid,
        in_specs=[pl.BlockSpec((BB, Kdim, L), lambda b: (b, 0, 0))],
        out_specs=pl.BlockSpec((BB, Kdim, L), lambda b: (b, 0, 0)),
        out_shape=jax.ShapeDtypeStruct((B, Kdim, L), x.dtype),
        compiler_params=pltpu.CompilerParams(
            dimension_semantics=("parallel",),
        ),
    )(x)
